# baseline (speedup 1.0000x reference)
import jax
import jax.numpy as jnp
from jax import lax
from jax.experimental import pallas as pl
from jax.experimental.pallas import tpu as pltpu
from jax.experimental.pallas import tpu_sc as plsc

MAX_NUM = 999999
D_MODEL = 32
QUAD = 128 // D_MODEL
NQUAD = (MAX_NUM + 1) // QUAD

NUM_CORES = 2
NUM_SUBCORES = 16
NUM_WORKERS = NUM_CORES * NUM_SUBCORES

NB = 8
HIST = 50
CHUNK = NB * HIST


def _sc_gather(table_pack, idx_flat, batch):
    b = idx_flat.shape[0]
    b_per_w = b // NUM_WORKERS
    nb_per_w = batch // NUM_WORKERS
    n_chunks = nb_per_w // NB
    mesh = plsc.VectorSubcoreMesh(core_axis_name="c", subcore_axis_name="s")

    @pl.kernel(
        out_type=jax.ShapeDtypeStruct((batch, HIST * D_MODEL), jnp.float32),
        mesh=mesh,
        scratch_types=[
            pltpu.VMEM((CHUNK,), jnp.int32),
            pltpu.VMEM((CHUNK,), jnp.int32),
            pltpu.VMEM((CHUNK, 128), jnp.float32),
            pltpu.VMEM((NB, HIST * D_MODEL), jnp.float32),
            pltpu.SemaphoreType.DMA,
        ],
    )
    def k(table_hbm, idx_hbm, out_hbm, idx_v, q_v, quad_v, stage_v, sem):
        wid = lax.axis_index("s") * NUM_CORES + lax.axis_index("c")
        base = wid * b_per_w
        bi_base = wid * nb_per_w

        @pl.loop(0, n_chunks)
        def _(g):
            off = pl.multiple_of(base + g * CHUNK, CHUNK)
            pltpu.sync_copy(idx_hbm.at[pl.ds(off, CHUNK)], idx_v)

            @pl.loop(0, CHUNK, step=16)
            def _(i):
                q_v[pl.ds(i, 16)] = jax.lax.shift_right_logical(
                    idx_v[pl.ds(i, 16)], 2
                )

            pltpu.async_copy(table_hbm.at[q_v], quad_v, sem).wait()

            @pl.loop(0, CHUNK, step=16)
            def _(r0):
                iv16 = idx_v[pl.ds(r0, 16)]
                for j in range(16):
                    r = r0 + j
                    src = (iv16[j] & 3) * D_MODEL
                    bb = r // HIST
                    dst = (r - bb * HIST) * D_MODEL
                    stage_v[bb, pl.ds(dst, 16)] = quad_v[r, pl.ds(src, 16)]
                    stage_v[bb, pl.ds(dst + 16, 16)] = quad_v[
                        r, pl.ds(src + 16, 16)
                    ]

            pltpu.sync_copy(stage_v, out_hbm.at[pl.ds(bi_base + g * NB, NB)])

    return k(table_pack, idx_flat)


FIN_BB = 256


def _tc_finish(out_flat, batch):
    def body(x_ref, o_ref):
        for h in range(HIST):
            o_ref[:, h, :] = x_ref[:, h * D_MODEL : (h + 1) * D_MODEL]

    return pl.pallas_call(
        body,
        grid=(batch // FIN_BB,),
        in_specs=[pl.BlockSpec((FIN_BB, HIST * D_MODEL), lambda i: (i, 0))],
        out_specs=pl.BlockSpec((FIN_BB, HIST, D_MODEL), lambda i: (i, 0, 0)),
        out_shape=jax.ShapeDtypeStruct((batch, HIST, D_MODEL), jnp.float32),
    )(out_flat)


def kernel(numbers, embed_table):
    batch, hist = numbers.shape
    idx_flat = numbers.reshape(batch * hist)
    table_pack = embed_table.reshape(NQUAD, D_MODEL * QUAD)
    out_flat = _sc_gather(table_pack, idx_flat, batch)
    return _tc_finish(out_flat, batch)


# speedup vs baseline: 1.8301x; 1.8301x over previous
import jax
import jax.numpy as jnp
from jax import lax
from jax.experimental import pallas as pl
from jax.experimental.pallas import tpu as pltpu
from jax.experimental.pallas import tpu_sc as plsc

MAX_NUM = 999999
D_MODEL = 32
QUAD = 128 // D_MODEL
NQUAD = (MAX_NUM + 1) // QUAD

NUM_CORES = 2
NUM_SUBCORES = 16
NUM_WORKERS = NUM_CORES * NUM_SUBCORES

NB = 8
HIST = 50
CHUNK = NB * HIST


def _sc_gather(table_pack, idx_flat, batch):
    b = idx_flat.shape[0]
    b_per_w = b // NUM_WORKERS
    nb_per_w = batch // NUM_WORKERS
    n_chunks = nb_per_w // NB
    mesh = plsc.VectorSubcoreMesh(core_axis_name="c", subcore_axis_name="s")

    @pl.kernel(
        out_type=jax.ShapeDtypeStruct((batch, HIST * D_MODEL), jnp.float32),
        mesh=mesh,
        scratch_types=[
            pltpu.VMEM((CHUNK,), jnp.int32),
            pltpu.VMEM((CHUNK,), jnp.int32),
            pltpu.VMEM((CHUNK, 128), jnp.float32),
            pltpu.VMEM((NB, HIST * D_MODEL), jnp.float32),
            pltpu.SemaphoreType.DMA,
        ],
    )
    def k(table_hbm, idx_hbm, out_hbm, idx_v, q_v, quad_v, stage_v, sem):
        wid = lax.axis_index("s") * NUM_CORES + lax.axis_index("c")
        base = wid * b_per_w
        bi_base = wid * nb_per_w

        @pl.loop(0, n_chunks)
        def _(g):
            off = pl.multiple_of(base + g * CHUNK, CHUNK)
            pltpu.sync_copy(idx_hbm.at[pl.ds(off, CHUNK)], idx_v)

            @pl.loop(0, CHUNK, step=16)
            def _(i):
                q_v[pl.ds(i, 16)] = jax.lax.shift_right_logical(
                    idx_v[pl.ds(i, 16)], 2
                )

            pltpu.async_copy(table_hbm.at[q_v], quad_v, sem).wait()

            @pl.loop(0, CHUNK, step=16)
            def _(r0):
                iv16 = idx_v[pl.ds(r0, 16)]
                for j in range(16):
                    r = r0 + j
                    src = (iv16[j] & 3) * D_MODEL
                    bb = r // HIST
                    dst = (r - bb * HIST) * D_MODEL
                    stage_v[bb, pl.ds(dst, 16)] = quad_v[r, pl.ds(src, 16)]
                    stage_v[bb, pl.ds(dst + 16, 16)] = quad_v[
                        r, pl.ds(src + 16, 16)
                    ]

            pltpu.sync_copy(stage_v, out_hbm.at[pl.ds(bi_base + g * NB, NB)])

    return k(table_pack, idx_flat)


FIN_BB = 256


def _tc_finish(out_flat, batch):
    def body(x_ref, o_ref):
        for h in range(HIST):
            o_ref[:, h, :] = x_ref[:, h * D_MODEL : (h + 1) * D_MODEL]

    return pl.pallas_call(
        body,
        grid=(batch // FIN_BB,),
        in_specs=[pl.BlockSpec((FIN_BB, HIST * D_MODEL), lambda i: (i, 0))],
        out_specs=pl.BlockSpec((FIN_BB, HIST, D_MODEL), lambda i: (i, 0, 0)),
        out_shape=jax.ShapeDtypeStruct((batch, HIST, D_MODEL), jnp.float32),
    )(out_flat)


def kernel(numbers, embed_table):
    batch, hist = numbers.shape
    idx_flat = numbers.reshape(batch * hist)
    table_pack = embed_table.reshape(NQUAD, D_MODEL * QUAD)
    out_flat = _sc_gather(table_pack, idx_flat, batch)
    return out_flat.reshape(batch, hist, D_MODEL)


# speedup vs baseline: 2.2508x; 1.2299x over previous
import jax
import jax.numpy as jnp
from jax import lax
from jax.experimental import pallas as pl
from jax.experimental.pallas import tpu as pltpu
from jax.experimental.pallas import tpu_sc as plsc

MAX_NUM = 999999
D_MODEL = 32
QUAD = 128 // D_MODEL
NQUAD = (MAX_NUM + 1) // QUAD

NUM_CORES = 2
NUM_SUBCORES = 16
NUM_WORKERS = NUM_CORES * NUM_SUBCORES

NB = 8
HIST = 50
CHUNK = NB * HIST


def _sc_gather(table_pack, idx_flat, batch):
    b = idx_flat.shape[0]
    b_per_w = b // NUM_WORKERS
    nb_per_w = batch // NUM_WORKERS
    n_chunks = nb_per_w // NB
    mesh = plsc.VectorSubcoreMesh(core_axis_name="c", subcore_axis_name="s")

    @pl.kernel(
        out_type=jax.ShapeDtypeStruct((batch, HIST * D_MODEL), jnp.float32),
        mesh=mesh,
        scratch_types=[
            pltpu.VMEM((CHUNK,), jnp.int32),
            pltpu.VMEM((CHUNK,), jnp.int32),
            pltpu.VMEM((CHUNK,), jnp.int32),
            pltpu.VMEM((CHUNK,), jnp.int32),
            pltpu.VMEM((CHUNK, 128), jnp.float32),
            pltpu.VMEM((CHUNK, 128), jnp.float32),
            pltpu.VMEM((NB, HIST * D_MODEL), jnp.float32),
            pltpu.VMEM((NB, HIST * D_MODEL), jnp.float32),
            pltpu.SemaphoreType.DMA,
            pltpu.SemaphoreType.DMA,
            pltpu.SemaphoreType.DMA,
            pltpu.SemaphoreType.DMA,
        ],
    )
    def k(
        table_hbm, idx_hbm, out_hbm,
        idx0, idx1, q0, q1, quad0, quad1, st0, st1,
        gsem0, gsem1, wsem0, wsem1,
    ):
        wid = lax.axis_index("s") * NUM_CORES + lax.axis_index("c")
        base = wid * b_per_w
        bi_base = wid * nb_per_w

        def start(g, idxb, qb, quadb, gsem):
            off = pl.multiple_of(base + (g % n_chunks) * CHUNK, 8)
            pltpu.sync_copy(idx_hbm.at[pl.ds(off, CHUNK)], idxb)

            @pl.loop(0, CHUNK, step=16)
            def _(i):
                qb[pl.ds(i, 16)] = jax.lax.shift_right_logical(
                    idxb[pl.ds(i, 16)], 2
                )

            pltpu.make_async_copy(table_hbm.at[qb], quadb, gsem).start()

        def finish(g, idxb, qb, quadb, stb, gsem, wsem):
            pltpu.make_async_copy(table_hbm.at[qb], quadb, gsem).wait()
            bi = pl.multiple_of(bi_base + g * NB, NB)
            wb = pltpu.make_async_copy(stb, out_hbm.at[pl.ds(bi, NB)], wsem)
            wb.wait()

            @pl.loop(0, CHUNK, step=16)
            def _(r0):
                iv16 = idxb[pl.ds(r0, 16)]
                for j in range(16):
                    r = r0 + j
                    src = (iv16[j] & 3) * D_MODEL
                    bb = r // HIST
                    dst = (r - bb * HIST) * D_MODEL
                    stb[bb, pl.ds(dst, 16)] = quadb[r, pl.ds(src, 16)]
                    stb[bb, pl.ds(dst + 16, 16)] = quadb[r, pl.ds(src + 16, 16)]

            wb.start()

        start(0, idx0, q0, quad0, gsem0)
        start(1, idx1, q1, quad1, gsem1)
        pltpu.make_async_copy(
            st0, out_hbm.at[pl.ds(pl.multiple_of(bi_base, NB), NB)], wsem0
        ).start()
        pltpu.make_async_copy(
            st1, out_hbm.at[pl.ds(pl.multiple_of(bi_base + NB, NB), NB)], wsem1
        ).start()

        @pl.loop(0, n_chunks, step=2)
        def _(g):
            finish(g, idx0, q0, quad0, st0, gsem0, wsem0)
            start(g + 2, idx0, q0, quad0, gsem0)
            finish(g + 1, idx1, q1, quad1, st1, gsem1, wsem1)
            start(g + 3, idx1, q1, quad1, gsem1)

        pltpu.make_async_copy(table_hbm.at[q0], quad0, gsem0).wait()
        pltpu.make_async_copy(table_hbm.at[q1], quad1, gsem1).wait()
        pltpu.make_async_copy(
            st0, out_hbm.at[pl.ds(pl.multiple_of(bi_base, NB), NB)], wsem0
        ).wait()
        pltpu.make_async_copy(
            st1, out_hbm.at[pl.ds(pl.multiple_of(bi_base + NB, NB), NB)], wsem1
        ).wait()

    return k(table_pack, idx_flat)


FIN_BB = 256


def _tc_finish(out_flat, batch):
    def body(x_ref, o_ref):
        for h in range(HIST):
            o_ref[:, h, :] = x_ref[:, h * D_MODEL : (h + 1) * D_MODEL]

    return pl.pallas_call(
        body,
        grid=(batch // FIN_BB,),
        in_specs=[pl.BlockSpec((FIN_BB, HIST * D_MODEL), lambda i: (i, 0))],
        out_specs=pl.BlockSpec((FIN_BB, HIST, D_MODEL), lambda i: (i, 0, 0)),
        out_shape=jax.ShapeDtypeStruct((batch, HIST, D_MODEL), jnp.float32),
    )(out_flat)


def kernel(numbers, embed_table):
    batch, hist = numbers.shape
    idx_flat = numbers.reshape(batch * hist)
    table_pack = embed_table.reshape(NQUAD, D_MODEL * QUAD)
    out_flat = _sc_gather(table_pack, idx_flat, batch)
    return out_flat.reshape(batch, hist, D_MODEL)
